# baseline (speedup 1.0000x reference)
import jax
import jax.numpy as jnp
from jax import lax
from jax.experimental import pallas as pl
from jax.experimental.pallas import tpu as pltpu, tpu_sc as plsc
import functools

B = 16384
VOCAB = 1000000
EMB = 16
NSPARSE = 26
NDENSE = 13
NFEAT = NSPARSE + NDENSE

NC = 2
NS = 16
NW = NC * NS
PER_W = B // NW
NB = 128
NH = 2
HB = NB // NH
HL = HB * NSPARSE
NCHUNK = PER_W // NB
NTILE = B // NB
FE = NFEAT * 2
TILE_W = 8 * NB

VPAD = 1024000
NBAND = 2
NCT = VPAD // 128
PT = 25
PSTEP = NCT // PT // NW

_mesh = plsc.VectorSubcoreMesh(core_axis_name="c", subcore_axis_name="s")

_sc_params = pltpu.CompilerParams(needs_layout_passes=False,
                                  use_tc_tiling_on_sc=False)


@functools.partial(
    pl.kernel,
    out_type=jax.ShapeDtypeStruct((VPAD, EMB), jnp.float32),
    mesh=_mesh,
    scratch_types=[
        pltpu.VMEM((NBAND * PT * 8 * 128,), jnp.float32),
        pltpu.VMEM((PT * 128, EMB), jnp.float32),
        pltpu.SemaphoreType.DMA,
    ],
    compiler_params=_sc_params,
)
def _pack_sc(t4_hbm, out_hbm, blk_v, row_v, sem):
    wid = lax.axis_index("s") * NC + lax.axis_index("c")
    iota16 = jnp.arange(16, dtype=jnp.int32)
    pat = (iota16 // 8) * (PT * 1024) + (iota16 % 8) * 128

    def step(k, _):
        tc0 = (wid * PSTEP + k) * PT
        cps = [pltpu.async_copy(
            t4_hbm.at[band, pl.ds(tc0 * 1024, PT * 1024)],
            blk_v.at[pl.ds(band * PT * 1024, PT * 1024)], sem)
            for band in range(NBAND)]
        for cp in cps:
            cp.wait()

        def one_row(j, _):
            off = j + lax.shift_right_logical(j, 7) * 896
            row_v[j, :] = plsc.load_gather(blk_v, [pat + off])
            return 0

        lax.fori_loop(0, PT * 128, one_row, 0, unroll=8)
        pltpu.sync_copy(row_v, out_hbm.at[pl.ds(tc0 * 128, PT * 128)])
        return 0

    lax.fori_loop(0, PSTEP, step, 0)


@functools.partial(
    pl.kernel,
    out_type=(
        jax.ShapeDtypeStruct((B,), jnp.float32),
        jax.ShapeDtypeStruct((B,), jnp.float32),
        jax.ShapeDtypeStruct((FE, NTILE, TILE_W), jnp.float32),
    ),
    mesh=_mesh,
    scratch_types=[
        pltpu.VMEM((NB * NSPARSE,), jnp.int32),
        pltpu.VMEM((NB, 16), jnp.float32),
        pltpu.VMEM((HL, EMB), jnp.float32),
        pltpu.VMEM((NB * NSPARSE,), jnp.float32),
        pltpu.VMEM((FE * TILE_W,), jnp.float32),
        pltpu.VMEM((NB, EMB), jnp.float32),
        pltpu.VMEM((NB, EMB), jnp.float32),
        pltpu.VMEM((NB,), jnp.float32),
        pltpu.VMEM((NB,), jnp.float32),
        pltpu.VMEM((NDENSE, EMB), jnp.float32),
        pltpu.VMEM((16,), jnp.float32),
        pltpu.SemaphoreType.DMA,
        pltpu.SemaphoreType.DMA,
        pltpu.SemaphoreType.DMA,
    ],
    compiler_params=_sc_params,
)
def _fm_sc(idx_hbm, dense_hbm, emb1_hbm, emb_hbm, w1_hbm, dw_hbm,
           y1_hbm, y2_hbm, feat_hbm,
           idx_v, dense_v, sp_v, one_v, feat_v, s_v, q_v,
           y1_v, y2_v, dw_v, w1_v, sem, sem1, semo):
    wid = lax.axis_index("s") * NC + lax.axis_index("c")
    pltpu.sync_copy(dw_hbm, dw_v)
    pltpu.sync_copy(w1_hbm, w1_v)

    dwregs = [dw_v[d, :] for d in range(NDENSE)]
    w1regs = w1_v[:]
    w1s = [w1regs[d] for d in range(NDENSE)]
    sw2 = []
    for d in range(NDENSE):
        w = dwregs[d]
        acc = w[0] * w[0]
        for e in range(1, EMB):
            acc = acc + w[e] * w[e]
        sw2.append(acc)
    iota16 = jnp.arange(16, dtype=jnp.int32)
    e_pat = iota16 * NB

    def chunk_body(c, _):
        base = wid * PER_W + c * NB
        tc = wid * NCHUNK + c

        pltpu.sync_copy(idx_hbm.at[wid, c], idx_v)
        cp_one = pltpu.async_copy(emb1_hbm.at[idx_v], one_v, sem1)

        def half_body(h, _):
            cp_emb = pltpu.async_copy(
                emb_hbm.at[idx_v.at[pl.ds(h * HL, HL)]], sp_v, sem)
            cp_emb.wait()

            def row_body(b, _):
                bl = h * HB + b
                b26 = b * NSPARSE
                s = jnp.zeros((16,), jnp.float32)
                q = jnp.zeros((16,), jnp.float32)
                wv = e_pat + bl
                for f in range(NSPARSE):
                    v = sp_v[b26 + f, :]
                    plsc.store_scatter(feat_v, [wv + (f * 2 * TILE_W)], v)
                    s = s + v
                    q = q + v * v
                s_v[bl, :] = s
                q_v[bl, :] = q
                return 0

            lax.fori_loop(0, HB, row_body, 0)
            return 0

        lax.fori_loop(0, NH, half_body, 0)
        cp_one.wait()

        pltpu.sync_copy(dense_hbm.at[wid, c], dense_v)

        def grp_body(j, _):
            lanes = pl.ds(16 * j, 16)
            b_vec = iota16 + 16 * j
            se = [plsc.load_gather(s_v, [b_vec, jnp.full((16,), e, jnp.int32)])
                  for e in range(EMB)]
            qsum = plsc.load_gather(q_v, [b_vec, jnp.zeros((16,), jnp.int32)])
            for e in range(1, EMB):
                qsum = qsum + plsc.load_gather(
                    q_v, [b_vec, jnp.full((16,), e, jnp.int32)])
            b26_vec = b_vec * NSPARSE
            acc1 = plsc.load_gather(one_v, [b26_vec])
            for f in range(1, NSPARSE):
                acc1 = acc1 + plsc.load_gather(one_v, [b26_vec + f])
            for d in range(NDENSE):
                x = plsc.load_gather(dense_v,
                                     [b_vec, jnp.full((16,), d, jnp.int32)])
                xx = x * x
                qsum = qsum + xx * sw2[d]
                acc1 = acc1 + w1s[d] * x
                w = dwregs[d]
                base_d = b_vec + ((NSPARSE + d) * 2 * TILE_W)
                for e in range(EMB):
                    val = x * w[e]
                    se[e] = se[e] + val
                    plsc.store_scatter(feat_v, [base_d + (e * NB)], val)
            ssum = se[0] * se[0]
            for e in range(1, EMB):
                ssum = ssum + se[e] * se[e]
            y1_v[lanes] = acc1
            y2_v[lanes] = 0.5 * (ssum - qsum)
            return 0

        lax.fori_loop(0, NB // 16, grp_body, 0)

        copies = [pltpu.async_copy(feat_v.at[pl.ds(fe * TILE_W, TILE_W)],
                                   feat_hbm.at[fe, tc], semo)
                  for fe in range(FE)]
        for cp in copies:
            cp.wait()
        pltpu.sync_copy(y1_v, y1_hbm.at[pl.ds(base, NB)])
        pltpu.sync_copy(y2_v, y2_hbm.at[pl.ds(base, NB)])
        return 0

    lax.fori_loop(0, NCHUNK, chunk_body, 0)


def kernel(sparse_inputs, dense_inputs, emb_one_table, emb_table,
           dense_w_one, dense_w):
    idx = sparse_inputs.reshape(NW, NCHUNK, NB * NSPARSE)
    dense_pad = jnp.pad(dense_inputs, ((0, 0), (0, 16 - NDENSE)))
    dense_r = dense_pad.reshape(NW, NCHUNK, NB, 16)
    emb1_flat = emb_one_table.reshape(VOCAB)
    dw = dense_w.reshape(NDENSE, EMB)
    w1 = jnp.pad(dense_w_one, (0, 16 - NDENSE))

    emb_t = jnp.pad(jnp.transpose(emb_table, (1, 0)),
                    ((0, 0), (0, VPAD - VOCAB)))
    t4 = (emb_t.reshape(NBAND, 8, NCT, 128).transpose(0, 2, 1, 3)
          .reshape(NBAND, NCT * 8 * 128))
    emb_lin = _pack_sc(t4)

    y1, y2, feat_phys = _fm_sc(idx, dense_r, emb1_flat, emb_lin, w1, dw)
    feat = (feat_phys.reshape(NFEAT, 2, NTILE, 8, NB)
            .transpose(2, 4, 0, 1, 3).reshape(B, NFEAT, EMB))
    return (y1.reshape(B, 1), y2.reshape(B, 1), feat)


# speedup vs baseline: 1.0022x; 1.0022x over previous
import jax
import jax.numpy as jnp
from jax import lax
from jax.experimental import pallas as pl
from jax.experimental.pallas import tpu as pltpu, tpu_sc as plsc
import functools

B = 16384
VOCAB = 1000000
EMB = 16
NSPARSE = 26
NDENSE = 13
NFEAT = NSPARSE + NDENSE

NC = 2
NS = 16
NW = NC * NS
PER_W = B // NW
NB = 128
NH = 2
HB = NB // NH
HL = HB * NSPARSE
NCHUNK = PER_W // NB
NTILE = B // NB
FE = NFEAT * 2
TILE_W = 8 * NB

VPAD = 1024000
NBAND = 2
NCT = VPAD // 128
PT = 25
PSTEP = NCT // PT // NW

_mesh = plsc.VectorSubcoreMesh(core_axis_name="c", subcore_axis_name="s")

_sc_params = pltpu.CompilerParams(needs_layout_passes=False,
                                  use_tc_tiling_on_sc=False)


@functools.partial(
    pl.kernel,
    out_type=jax.ShapeDtypeStruct((VPAD, EMB), jnp.float32),
    mesh=_mesh,
    scratch_types=[
        pltpu.VMEM((NBAND * PT * 8 * 128,), jnp.float32),
        pltpu.VMEM((PT * 128, EMB), jnp.float32),
        pltpu.SemaphoreType.DMA,
    ],
    compiler_params=_sc_params,
)
def _pack_sc(t4_hbm, out_hbm, blk_v, row_v, sem):
    wid = lax.axis_index("s") * NC + lax.axis_index("c")
    iota16 = jnp.arange(16, dtype=jnp.int32)
    pat = (iota16 // 8) * (PT * 1024) + (iota16 % 8) * 128

    def step(k, _):
        tc0 = (wid * PSTEP + k) * PT
        cps = [pltpu.async_copy(
            t4_hbm.at[band, pl.ds(tc0 * 1024, PT * 1024)],
            blk_v.at[pl.ds(band * PT * 1024, PT * 1024)], sem)
            for band in range(NBAND)]
        for cp in cps:
            cp.wait()

        def one_row(j, _):
            off = j + lax.shift_right_logical(j, 7) * 896
            row_v[j, :] = plsc.load_gather(blk_v, [pat + off])
            return 0

        lax.fori_loop(0, PT * 128, one_row, 0, unroll=8)
        pltpu.sync_copy(row_v, out_hbm.at[pl.ds(tc0 * 128, PT * 128)])
        return 0

    lax.fori_loop(0, PSTEP, step, 0)


@functools.partial(
    pl.kernel,
    out_type=(
        jax.ShapeDtypeStruct((B,), jnp.float32),
        jax.ShapeDtypeStruct((B,), jnp.float32),
        jax.ShapeDtypeStruct((FE, NTILE, TILE_W), jnp.float32),
    ),
    mesh=_mesh,
    scratch_types=[
        pltpu.VMEM((NB * NSPARSE,), jnp.int32),
        pltpu.VMEM((NB, 17), jnp.float32),
        pltpu.VMEM((HL, EMB), jnp.float32),
        pltpu.VMEM((NB * NSPARSE,), jnp.float32),
        pltpu.VMEM((FE * TILE_W,), jnp.float32),
        pltpu.VMEM((NB, 17), jnp.float32),
        pltpu.VMEM((NB, 17), jnp.float32),
        pltpu.VMEM((NB,), jnp.float32),
        pltpu.VMEM((NB,), jnp.float32),
        pltpu.VMEM((NDENSE, EMB), jnp.float32),
        pltpu.VMEM((16,), jnp.float32),
        pltpu.SemaphoreType.DMA,
        pltpu.SemaphoreType.DMA,
        pltpu.SemaphoreType.DMA,
    ],
    compiler_params=_sc_params,
)
def _fm_sc(idx_hbm, dense_hbm, emb1_hbm, emb_hbm, w1_hbm, dw_hbm,
           y1_hbm, y2_hbm, feat_hbm,
           idx_v, dense_v, sp_v, one_v, feat_v, s_v, q_v,
           y1_v, y2_v, dw_v, w1_v, sem, sem1, semo):
    wid = lax.axis_index("s") * NC + lax.axis_index("c")
    pltpu.sync_copy(dw_hbm, dw_v)
    pltpu.sync_copy(w1_hbm, w1_v)

    dwregs = [dw_v[d, :] for d in range(NDENSE)]
    w1regs = w1_v[:]
    w1s = [w1regs[d] for d in range(NDENSE)]
    sw2 = []
    for d in range(NDENSE):
        w = dwregs[d]
        acc = w[0] * w[0]
        for e in range(1, EMB):
            acc = acc + w[e] * w[e]
        sw2.append(acc)
    iota16 = jnp.arange(16, dtype=jnp.int32)
    e_pat = iota16 * NB

    def chunk_body(c, _):
        base = wid * PER_W + c * NB
        tc = wid * NCHUNK + c

        pltpu.sync_copy(idx_hbm.at[wid, c], idx_v)
        cp_one = pltpu.async_copy(emb1_hbm.at[idx_v], one_v, sem1)

        def half_body(h, _):
            cp_emb = pltpu.async_copy(
                emb_hbm.at[idx_v.at[pl.ds(h * HL, HL)]], sp_v, sem)
            cp_emb.wait()

            def row_body(b, _):
                bl = h * HB + b
                b26 = b * NSPARSE
                s = jnp.zeros((16,), jnp.float32)
                q = jnp.zeros((16,), jnp.float32)
                wv = e_pat + bl
                for f in range(NSPARSE):
                    v = sp_v[b26 + f, :]
                    plsc.store_scatter(feat_v, [wv + (f * 2 * TILE_W)], v)
                    s = s + v
                    q = q + v * v
                s_v[bl, pl.ds(0, 16)] = s
                q_v[bl, pl.ds(0, 16)] = q
                return 0

            lax.fori_loop(0, HB, row_body, 0)
            return 0

        lax.fori_loop(0, NH, half_body, 0)
        cp_one.wait()

        pltpu.sync_copy(dense_hbm.at[wid, c], dense_v)

        def grp_body(j, _):
            lanes = pl.ds(16 * j, 16)
            b_vec = iota16 + 16 * j
            se = [plsc.load_gather(s_v, [b_vec, jnp.full((16,), e, jnp.int32)])
                  for e in range(EMB)]
            qsum = plsc.load_gather(q_v, [b_vec, jnp.zeros((16,), jnp.int32)])
            for e in range(1, EMB):
                qsum = qsum + plsc.load_gather(
                    q_v, [b_vec, jnp.full((16,), e, jnp.int32)])
            b26_vec = b_vec * NSPARSE
            acc1 = plsc.load_gather(one_v, [b26_vec])
            for f in range(1, NSPARSE):
                acc1 = acc1 + plsc.load_gather(one_v, [b26_vec + f])
            for d in range(NDENSE):
                x = plsc.load_gather(dense_v,
                                     [b_vec, jnp.full((16,), d, jnp.int32)])
                xx = x * x
                qsum = qsum + xx * sw2[d]
                acc1 = acc1 + w1s[d] * x
                w = dwregs[d]
                base_d = b_vec + ((NSPARSE + d) * 2 * TILE_W)
                for e in range(EMB):
                    val = x * w[e]
                    se[e] = se[e] + val
                    plsc.store_scatter(feat_v, [base_d + (e * NB)], val)
            ssum = se[0] * se[0]
            for e in range(1, EMB):
                ssum = ssum + se[e] * se[e]
            y1_v[lanes] = acc1
            y2_v[lanes] = 0.5 * (ssum - qsum)
            return 0

        lax.fori_loop(0, NB // 16, grp_body, 0)

        copies = [pltpu.async_copy(feat_v.at[pl.ds(fe * TILE_W, TILE_W)],
                                   feat_hbm.at[fe, tc], semo)
                  for fe in range(FE)]
        for cp in copies:
            cp.wait()
        pltpu.sync_copy(y1_v, y1_hbm.at[pl.ds(base, NB)])
        pltpu.sync_copy(y2_v, y2_hbm.at[pl.ds(base, NB)])
        return 0

    lax.fori_loop(0, NCHUNK, chunk_body, 0)


def kernel(sparse_inputs, dense_inputs, emb_one_table, emb_table,
           dense_w_one, dense_w):
    idx = sparse_inputs.reshape(NW, NCHUNK, NB * NSPARSE)
    dense_pad = jnp.pad(dense_inputs, ((0, 0), (0, 17 - NDENSE)))
    dense_r = dense_pad.reshape(NW, NCHUNK, NB, 17)
    emb1_flat = emb_one_table.reshape(VOCAB)
    dw = dense_w.reshape(NDENSE, EMB)
    w1 = jnp.pad(dense_w_one, (0, 16 - NDENSE))

    emb_t = jnp.pad(jnp.transpose(emb_table, (1, 0)),
                    ((0, 0), (0, VPAD - VOCAB)))
    t4 = (emb_t.reshape(NBAND, 8, NCT, 128).transpose(0, 2, 1, 3)
          .reshape(NBAND, NCT * 8 * 128))
    emb_lin = _pack_sc(t4)

    y1, y2, feat_phys = _fm_sc(idx, dense_r, emb1_flat, emb_lin, w1, dw)
    feat = (feat_phys.reshape(NFEAT, 2, NTILE, 8, NB)
            .transpose(2, 4, 0, 1, 3).reshape(B, NFEAT, EMB))
    return (y1.reshape(B, 1), y2.reshape(B, 1), feat)


# speedup vs baseline: 1.0513x; 1.0490x over previous
import jax
import jax.numpy as jnp
from jax import lax
from jax.experimental import pallas as pl
from jax.experimental.pallas import tpu as pltpu, tpu_sc as plsc
import functools

B = 16384
VOCAB = 1000000
EMB = 16
NSPARSE = 26
NDENSE = 13
NFEAT = NSPARSE + NDENSE

NC = 2
NS = 16
NW = NC * NS
PER_W = B // NW
NB = 128
NH = 2
HB = NB // NH
HL = HB * NSPARSE
NCHUNK = PER_W // NB
NTILE = B // NB
FE = NFEAT * 2
TILE_W = 8 * NB

VPAD = 1024000
NBAND = 2
NCT = VPAD // 128
PT = 25
PSTEP = NCT // PT // NW

_mesh = plsc.VectorSubcoreMesh(core_axis_name="c", subcore_axis_name="s")

_sc_params = pltpu.CompilerParams(needs_layout_passes=False,
                                  use_tc_tiling_on_sc=False)


@functools.partial(
    pl.kernel,
    out_type=jax.ShapeDtypeStruct((VPAD, EMB), jnp.float32),
    mesh=_mesh,
    scratch_types=[
        pltpu.VMEM((NBAND * PT * 8 * 128,), jnp.float32),
        pltpu.VMEM((PT * 128, EMB), jnp.float32),
        pltpu.SemaphoreType.DMA,
    ],
    compiler_params=_sc_params,
)
def _pack_sc(t4_hbm, out_hbm, blk_v, row_v, sem):
    wid = lax.axis_index("s") * NC + lax.axis_index("c")
    iota16 = jnp.arange(16, dtype=jnp.int32)
    pat = (iota16 // 8) * (PT * 1024) + (iota16 % 8) * 128

    def step(k, _):
        tc0 = (wid * PSTEP + k) * PT
        cps = [pltpu.async_copy(
            t4_hbm.at[band, pl.ds(tc0 * 1024, PT * 1024)],
            blk_v.at[pl.ds(band * PT * 1024, PT * 1024)], sem)
            for band in range(NBAND)]
        for cp in cps:
            cp.wait()

        def one_row(j, _):
            off = j + lax.shift_right_logical(j, 7) * 896
            row_v[j, :] = plsc.load_gather(blk_v, [pat + off])
            return 0

        lax.fori_loop(0, PT * 128, one_row, 0, unroll=8)
        pltpu.sync_copy(row_v, out_hbm.at[pl.ds(tc0 * 128, PT * 128)])
        return 0

    lax.fori_loop(0, PSTEP, step, 0)


@functools.partial(
    pl.kernel,
    out_type=(
        jax.ShapeDtypeStruct((B,), jnp.float32),
        jax.ShapeDtypeStruct((B,), jnp.float32),
        jax.ShapeDtypeStruct((FE, NTILE, TILE_W), jnp.float32),
    ),
    mesh=_mesh,
    scratch_types=[
        pltpu.VMEM((NB * NSPARSE,), jnp.int32),
        pltpu.VMEM((NB, 17), jnp.float32),
        pltpu.VMEM((HL, EMB), jnp.float32),
        pltpu.VMEM((NB * NSPARSE,), jnp.float32),
        pltpu.VMEM((FE * TILE_W,), jnp.float32),
        pltpu.VMEM((NB, 17), jnp.float32),
        pltpu.VMEM((NB, 17), jnp.float32),
        pltpu.VMEM((NB,), jnp.float32),
        pltpu.VMEM((NB,), jnp.float32),
        pltpu.VMEM((NDENSE, EMB), jnp.float32),
        pltpu.VMEM((16,), jnp.float32),
        pltpu.SemaphoreType.DMA,
        pltpu.SemaphoreType.DMA,
        pltpu.SemaphoreType.DMA,
    ],
    compiler_params=_sc_params,
)
def _fm_sc(idx_hbm, dense_hbm, emb1_hbm, emb_hbm, w1_hbm, dw_hbm,
           y1_hbm, y2_hbm, feat_hbm,
           idx_v, dense_v, sp_v, one_v, feat_v, s_v, q_v,
           y1_v, y2_v, dw_v, w1_v, sem, sem1, semo):
    wid = lax.axis_index("s") * NC + lax.axis_index("c")
    pltpu.sync_copy(dw_hbm, dw_v)
    pltpu.sync_copy(w1_hbm, w1_v)

    dwregs = [dw_v[d, :] for d in range(NDENSE)]
    w1regs = w1_v[:]
    w1s = [w1regs[d] for d in range(NDENSE)]
    sw2 = []
    for d in range(NDENSE):
        w = dwregs[d]
        acc = w[0] * w[0]
        for e in range(1, EMB):
            acc = acc + w[e] * w[e]
        sw2.append(acc)
    iota16 = jnp.arange(16, dtype=jnp.int32)
    e_pat = iota16 * NB

    def chunk_body(c, _):
        base = wid * PER_W + c * NB
        tc = wid * NCHUNK + c

        pltpu.sync_copy(idx_hbm.at[wid, c], idx_v)
        cp_one = pltpu.async_copy(emb1_hbm.at[idx_v], one_v, sem1)

        def half_body(h, _):
            cp_emb = pltpu.async_copy(
                emb_hbm.at[idx_v.at[pl.ds(h * HL, HL)]], sp_v, sem)
            cp_emb.wait()

            def row_body(b, _):
                bl = h * HB + b
                b26 = b * NSPARSE
                s = jnp.zeros((16,), jnp.float32)
                q = jnp.zeros((16,), jnp.float32)
                wv = e_pat + bl
                for f in range(NSPARSE):
                    v = sp_v[b26 + f, :]
                    plsc.store_scatter(feat_v, [wv + (f * 2 * TILE_W)], v)
                    s = s + v
                    q = q + v * v
                s_v[bl, pl.ds(0, 16)] = s
                q_v[bl, pl.ds(0, 16)] = q
                return 0

            lax.fori_loop(0, HB, row_body, 0)
            return 0

        lax.fori_loop(0, NH, half_body, 0)
        cp_one.wait()

        pltpu.sync_copy(dense_hbm.at[wid, c], dense_v)

        def grp_body(j, _):
            lanes = pl.ds(16 * j, 16)
            b_vec = iota16 + 16 * j
            se = [plsc.load_gather(s_v, [b_vec, jnp.full((16,), e, jnp.int32)])
                  for e in range(EMB)]
            qsum = plsc.load_gather(q_v, [b_vec, jnp.zeros((16,), jnp.int32)])
            for e in range(1, EMB):
                qsum = qsum + plsc.load_gather(
                    q_v, [b_vec, jnp.full((16,), e, jnp.int32)])
            b26_vec = b_vec * NSPARSE
            acc1 = plsc.load_gather(one_v, [b26_vec])
            for f in range(1, NSPARSE):
                acc1 = acc1 + plsc.load_gather(one_v, [b26_vec + f])
            for d in range(NDENSE):
                x = plsc.load_gather(dense_v,
                                     [b_vec, jnp.full((16,), d, jnp.int32)])
                xx = x * x
                qsum = qsum + xx * sw2[d]
                acc1 = acc1 + w1s[d] * x
                w = dwregs[d]
                base_d = b_vec + ((NSPARSE + d) * 2 * TILE_W)
                for e in range(EMB):
                    val = x * w[e]
                    se[e] = se[e] + val
                    plsc.store_scatter(feat_v, [base_d + (e * NB)], val)
            ssum = se[0] * se[0]
            for e in range(1, EMB):
                ssum = ssum + se[e] * se[e]
            y1_v[lanes] = acc1
            y2_v[lanes] = 0.5 * (ssum - qsum)
            return 0

        lax.fori_loop(0, NB // 16, grp_body, 0)

        copies = [pltpu.async_copy(feat_v.at[pl.ds(fe * TILE_W, TILE_W)],
                                   feat_hbm.at[fe, tc], semo)
                  for fe in range(FE)]
        for cp in copies:
            cp.wait()
        pltpu.sync_copy(y1_v, y1_hbm.at[pl.ds(base, NB)])
        pltpu.sync_copy(y2_v, y2_hbm.at[pl.ds(base, NB)])
        return 0

    lax.fori_loop(0, NCHUNK, chunk_body, 0)


def kernel(sparse_inputs, dense_inputs, emb_one_table, emb_table,
           dense_w_one, dense_w):
    idx = sparse_inputs.reshape(NW, NCHUNK, NB * NSPARSE)
    dense_pad = jnp.pad(dense_inputs, ((0, 0), (0, 17 - NDENSE)))
    dense_r = dense_pad.reshape(NW, NCHUNK, NB, 17)
    emb1_flat = emb_one_table.reshape(VOCAB)
    dw = dense_w.reshape(NDENSE, EMB)
    w1 = jnp.pad(dense_w_one, (0, 16 - NDENSE))

    y1, y2, feat_phys = _fm_sc(idx, dense_r, emb1_flat, emb_table, w1, dw)
    feat = (feat_phys.reshape(NFEAT, 2, NTILE, 8, NB)
            .transpose(2, 4, 0, 1, 3).reshape(B, NFEAT, EMB))
    return (y1.reshape(B, 1), y2.reshape(B, 1), feat)
